# Initial kernel scaffold; baseline (speedup 1.0000x reference)
import jax
import jax.numpy as jnp
from jax.experimental import pallas as pl


def kernel(embeddings, spans, W, b):
    raise NotImplementedError("write your pallas kernel here")


# speedup vs baseline: 1.7856x; 1.7856x over previous
import jax
import jax.numpy as jnp
from jax import lax
from jax.experimental import pallas as pl
from jax.experimental.pallas import tpu as pltpu, tpu_sc as plsc

NC, NS, L = 2, 16, 16
NW = NC * NS
DIM = 1024
NSPANS = 16 * 512
PER_W = NSPANS // NW
CH = 16
NCHUNK = PER_W // CH
ODIM = 2 * DIM + 3


def _sc_body(emb, sidx, eidx, out,
             sidx_v, eidx_v, s0, s1, e0, e1,
             sem_s0, sem_s1, sem_e0, sem_e1):
    wid = lax.axis_index("s") * NC + lax.axis_index("c")
    base = wid * PER_W

    pltpu.sync_copy(sidx.at[pl.ds(base, PER_W)], sidx_v)
    pltpu.sync_copy(eidx.at[pl.ds(base, PER_W)], eidx_v)

    srows, erows = [s0, s1], [e0, e1]
    sem_s, sem_e = [sem_s0, sem_s1], [sem_e0, sem_e1]

    def issue(g):
        slot = g & 1
        cs = pltpu.async_copy(emb.at[sidx_v.at[pl.ds(g * CH, CH)]],
                              srows[slot], sem_s[slot])
        ce = pltpu.async_copy(emb.at[eidx_v.at[pl.ds(g * CH, CH)]],
                              erows[slot], sem_e[slot])
        return cs, ce

    pend = issue(0)
    for g in range(NCHUNK):
        nxt = issue(g + 1) if g + 1 < NCHUNK else None
        slot = g & 1
        cs, ce = pend
        cs.wait()
        ce.wait()
        gbase = base + g * CH
        pltpu.sync_copy(srows[slot], out.at[pl.ds(gbase, CH), pl.ds(0, DIM)])
        pltpu.sync_copy(erows[slot], out.at[pl.ds(gbase, CH), pl.ds(DIM, DIM)])
        pend = nxt


def _make_sc_call():
    mesh = plsc.VectorSubcoreMesh(core_axis_name="c", subcore_axis_name="s",
                                  num_cores=NC, num_subcores=NS)
    return pl.kernel(
        _sc_body,
        out_type=jax.ShapeDtypeStruct((NSPANS, ODIM), jnp.float32),
        mesh=mesh,
        scratch_types=[
            pltpu.VMEM((PER_W,), jnp.int32),
            pltpu.VMEM((PER_W,), jnp.int32),
            pltpu.VMEM((CH, DIM), jnp.float32),
            pltpu.VMEM((CH, DIM), jnp.float32),
            pltpu.VMEM((CH, DIM), jnp.float32),
            pltpu.VMEM((CH, DIM), jnp.float32),
            pltpu.SemaphoreType.DMA,
            pltpu.SemaphoreType.DMA,
            pltpu.SemaphoreType.DMA,
            pltpu.SemaphoreType.DMA,
        ],
        compiler_params=pltpu.CompilerParams(use_tc_tiling_on_sc=True),
        name="end_point_aggregator_sc",
    )


def _dist_body(s_ref, e_ref, wb_ref, se_ref, out_ref):
    del se_ref
    d = (e_ref[...] - s_ref[...]).astype(jnp.float32)
    col = lax.broadcasted_iota(jnp.int32, (1, 128), 1)
    w = jnp.where(col == 0, wb_ref[0, 0],
                  jnp.where(col == 1, wb_ref[0, 1], wb_ref[0, 2]))
    bb = jnp.where(col == 0, wb_ref[0, 3],
                   jnp.where(col == 1, wb_ref[0, 4], wb_ref[0, 5]))
    out_ref[...] = jnp.tanh(d * w + bb)


def _dist_call(sidx, eidx, wb, se):
    ncols = ODIM // 128
    return pl.pallas_call(
        _dist_body,
        out_shape=jax.ShapeDtypeStruct((NSPANS, ODIM), jnp.float32),
        grid=(1,),
        in_specs=[
            pl.BlockSpec((NSPANS, 1), lambda i: (0, 0)),
            pl.BlockSpec((NSPANS, 1), lambda i: (0, 0)),
            pl.BlockSpec(memory_space=pltpu.SMEM),
            pl.BlockSpec(memory_space=pl.ANY),
        ],
        out_specs=pl.BlockSpec((NSPANS, 128), lambda i: (0, ncols)),
        input_output_aliases={3: 0},
        name="end_point_aggregator_dist",
    )(sidx, eidx, wb, se)


def kernel(embeddings, spans, W, b):
    B, S, D = embeddings.shape
    n = spans.shape[1]
    spans_i = spans.astype(jnp.int32)
    offs = (jnp.arange(B, dtype=jnp.int32) * S)[:, None]
    sidx = (spans_i[..., 0] + offs).reshape(-1)
    eidx = (spans_i[..., 1] + offs).reshape(-1)
    emb = embeddings.reshape(B * S, D)
    se = _make_sc_call()(emb, sidx, eidx)
    wb = jnp.concatenate([W[:, 0], b]).reshape(1, 6)
    out = _dist_call(sidx[:, None], eidx[:, None], wb, se)
    return out.reshape(B, n, ODIM)


# speedup vs baseline: 1.7978x; 1.0068x over previous
import jax
import jax.numpy as jnp
from jax import lax
from jax.experimental import pallas as pl
from jax.experimental.pallas import tpu as pltpu, tpu_sc as plsc

NC, NS, L = 2, 16, 16
NW = NC * NS
DIM = 1024
NSPANS = 16 * 512
PER_W = NSPANS // NW
CH = 16
NCHUNK = PER_W // CH
ODIM = 2 * DIM + 3


NSLOT = 3
GAHEAD = 2


def _sc_body(emb, sidx, eidx, out,
             sidx_v, eidx_v, b0, b1, b2,
             gs0, gs1, gs2, ge0, ge1, ge2, ws0, ws1, ws2):
    wid = lax.axis_index("s") * NC + lax.axis_index("c")
    base = wid * PER_W

    pltpu.sync_copy(sidx.at[pl.ds(base, PER_W)], sidx_v)
    pltpu.sync_copy(eidx.at[pl.ds(base, PER_W)], eidx_v)

    buf = [b0, b1, b2]
    sem_gs, sem_ge = [gs0, gs1, gs2], [ge0, ge1, ge2]
    sem_w = [ws0, ws1, ws2]

    def issue_gathers(g):
        slot = g % NSLOT
        cs = pltpu.async_copy(emb.at[sidx_v.at[pl.ds(g * CH, CH)]],
                              buf[slot].at[:, pl.ds(0, DIM)], sem_gs[slot])
        ce = pltpu.async_copy(emb.at[eidx_v.at[pl.ds(g * CH, CH)]],
                              buf[slot].at[:, pl.ds(DIM, DIM)], sem_ge[slot])
        return cs, ce

    gd = [None] * NCHUNK
    wd = [None] * NCHUNK
    for g in range(NCHUNK + GAHEAD):
        if g < NCHUNK:
            if g >= NSLOT:
                wd[g - NSLOT].wait()
            gd[g] = issue_gathers(g)
        h = g - GAHEAD
        if h >= 0:
            cs, ce = gd[h]
            cs.wait()
            ce.wait()
            slot = h % NSLOT
            wd[h] = pltpu.async_copy(
                buf[slot],
                out.at[pl.ds(base + h * CH, CH), pl.ds(0, 2 * DIM)],
                sem_w[slot])
    for h in range(NCHUNK - NSLOT, NCHUNK):
        wd[h].wait()


def _make_sc_call():
    mesh = plsc.VectorSubcoreMesh(core_axis_name="c", subcore_axis_name="s",
                                  num_cores=NC, num_subcores=NS)
    return pl.kernel(
        _sc_body,
        out_type=jax.ShapeDtypeStruct((NSPANS, ODIM), jnp.float32),
        mesh=mesh,
        scratch_types=[
            pltpu.VMEM((PER_W,), jnp.int32),
            pltpu.VMEM((PER_W,), jnp.int32),
            pltpu.VMEM((CH, 2 * DIM), jnp.float32),
            pltpu.VMEM((CH, 2 * DIM), jnp.float32),
            pltpu.VMEM((CH, 2 * DIM), jnp.float32),
            pltpu.SemaphoreType.DMA,
            pltpu.SemaphoreType.DMA,
            pltpu.SemaphoreType.DMA,
            pltpu.SemaphoreType.DMA,
            pltpu.SemaphoreType.DMA,
            pltpu.SemaphoreType.DMA,
            pltpu.SemaphoreType.DMA,
            pltpu.SemaphoreType.DMA,
            pltpu.SemaphoreType.DMA,
        ],
        compiler_params=pltpu.CompilerParams(use_tc_tiling_on_sc=True),
        name="end_point_aggregator_sc",
    )


def _dist_body(s_ref, e_ref, wb_ref, se_ref, out_ref):
    del se_ref
    d = (e_ref[...] - s_ref[...]).astype(jnp.float32)
    col = lax.broadcasted_iota(jnp.int32, (1, 128), 1)
    w = jnp.where(col == 0, wb_ref[0, 0],
                  jnp.where(col == 1, wb_ref[0, 1], wb_ref[0, 2]))
    bb = jnp.where(col == 0, wb_ref[0, 3],
                   jnp.where(col == 1, wb_ref[0, 4], wb_ref[0, 5]))
    out_ref[...] = jnp.tanh(d * w + bb)


def _dist_call(sidx, eidx, wb, se):
    ncols = ODIM // 128
    return pl.pallas_call(
        _dist_body,
        out_shape=jax.ShapeDtypeStruct((NSPANS, ODIM), jnp.float32),
        grid=(1,),
        in_specs=[
            pl.BlockSpec((NSPANS, 1), lambda i: (0, 0)),
            pl.BlockSpec((NSPANS, 1), lambda i: (0, 0)),
            pl.BlockSpec(memory_space=pltpu.SMEM),
            pl.BlockSpec(memory_space=pl.ANY),
        ],
        out_specs=pl.BlockSpec((NSPANS, 128), lambda i: (0, ncols)),
        input_output_aliases={3: 0},
        name="end_point_aggregator_dist",
    )(sidx, eidx, wb, se)


def kernel(embeddings, spans, W, b):
    B, S, D = embeddings.shape
    n = spans.shape[1]
    spans_i = spans.astype(jnp.int32)
    offs = (jnp.arange(B, dtype=jnp.int32) * S)[:, None]
    sidx = (spans_i[..., 0] + offs).reshape(-1)
    eidx = (spans_i[..., 1] + offs).reshape(-1)
    emb = embeddings.reshape(B * S, D)
    se = _make_sc_call()(emb, sidx, eidx)
    wb = jnp.concatenate([W[:, 0], b]).reshape(1, 6)
    out = _dist_call(sidx[:, None], eidx[:, None], wb, se)
    return out.reshape(B, n, ODIM)
